# Initial kernel scaffold; baseline (speedup 1.0000x reference)
import jax
import jax.numpy as jnp
from jax.experimental import pallas as pl


def kernel(x, edge_index, W1, b1, W2, b2, W3, b3):
    raise NotImplementedError("write your pallas kernel here")


# speedup vs baseline: 13.6623x; 13.6623x over previous
import functools

import jax
import jax.numpy as jnp
from jax import lax
from jax.experimental import pallas as pl
from jax.experimental.pallas import tpu as pltpu
from jax.experimental.pallas import tpu_sc as plsc

NC = 2
NS = 16
NW = NC * NS
K_EDGE = 80


def _ceil_to(a, b):
    return (a + b - 1) // b * b


def _sc_agg(N, NPAD, EP, D):
    EPT = EP // NW
    CH = EPT // K_EDGE
    RPS = NPAD // NS
    assert RPS % K_EDGE == 0 and EPT % K_EDGE == 0

    mesh = plsc.VectorSubcoreMesh(core_axis_name="c", subcore_axis_name="s",
                                  num_cores=NC, num_subcores=NS)

    @functools.partial(
        pl.kernel,
        out_type=jax.ShapeDtypeStruct((NC, NPAD, D), jnp.float32),
        mesh=mesh,
        scratch_types=[
            pltpu.VMEM((K_EDGE,), jnp.int32),
            pltpu.VMEM((K_EDGE,), jnp.int32),
            pltpu.VMEM((K_EDGE, D), jnp.float32),
            pltpu.VMEM_SHARED((NPAD, D), jnp.float32),
            pltpu.SemaphoreType.DMA,
        ],
        compiler_params=pltpu.CompilerParams(use_tc_tiling_on_sc=False),
    )
    def agg(table, src, dst, out, idx_s, idx_d, rows, acc, sem):
        c = lax.axis_index("c")
        s = lax.axis_index("s")
        zeros16 = jnp.zeros((16,), jnp.float32)

        def zrow(r, carry):
            for j in range(D // 16):
                rows[r, pl.ds(j * 16, 16)] = zeros16
            return carry

        lax.fori_loop(0, K_EDGE, zrow, 0)
        sub0 = s * RPS
        for j in range(RPS // K_EDGE):
            pltpu.sync_copy(rows, acc.at[pl.ds(sub0 + j * K_EDGE, K_EDGE)])
        plsc.subcore_barrier()

        ebase = (c * NS + s) * EPT

        def chunk(i, carry):
            b = ebase + i * K_EDGE
            pltpu.sync_copy(src.at[pl.ds(b, K_EDGE)], idx_s)
            pltpu.sync_copy(dst.at[pl.ds(b, K_EDGE)], idx_d)
            pltpu.async_copy(table.at[idx_s], rows, sem).wait()
            pltpu.sync_copy(rows, acc.at[idx_d], add=True)
            return carry

        lax.fori_loop(0, CH, chunk, 0)
        plsc.subcore_barrier()
        for j in range(RPS // K_EDGE):
            sl = pl.ds(sub0 + j * K_EDGE, K_EDGE)
            pltpu.sync_copy(acc.at[sl], out.at[c, sl])

    return agg


def _sc_deg(NPAD, EP):
    D = 16
    EPT = EP // NW
    CH = EPT // K_EDGE
    RPS = NPAD // NS
    assert RPS % K_EDGE == 0 and EPT % K_EDGE == 0

    mesh = plsc.VectorSubcoreMesh(core_axis_name="c", subcore_axis_name="s",
                                  num_cores=NC, num_subcores=NS)

    @functools.partial(
        pl.kernel,
        out_type=jax.ShapeDtypeStruct((NC, NPAD, D), jnp.float32),
        mesh=mesh,
        scratch_types=[
            pltpu.VMEM((K_EDGE,), jnp.int32),
            pltpu.VMEM((K_EDGE, D), jnp.float32),
            pltpu.VMEM_SHARED((NPAD, D), jnp.float32),
        ],
    )
    def deg(dst, out, idx_d, rows, acc):
        c = lax.axis_index("c")
        s = lax.axis_index("s")
        zeros16 = jnp.zeros((16,), jnp.float32)
        ones16 = jnp.ones((16,), jnp.float32)

        def fill(vec):
            def body(r, carry):
                rows[r, pl.ds(0, 16)] = vec
                return carry
            lax.fori_loop(0, K_EDGE, body, 0)

        fill(zeros16)
        sub0 = s * RPS
        for j in range(RPS // K_EDGE):
            pltpu.sync_copy(rows, acc.at[pl.ds(sub0 + j * K_EDGE, K_EDGE)])
        fill(ones16)
        plsc.subcore_barrier()

        ebase = (c * NS + s) * EPT

        def chunk(i, carry):
            b = ebase + i * K_EDGE
            pltpu.sync_copy(dst.at[pl.ds(b, K_EDGE)], idx_d)
            pltpu.sync_copy(rows, acc.at[idx_d], add=True)
            return carry

        lax.fori_loop(0, CH, chunk, 0)
        plsc.subcore_barrier()
        for j in range(RPS // K_EDGE):
            sl = pl.ds(sub0 + j * K_EDGE, K_EDGE)
            pltpu.sync_copy(acc.at[sl], out.at[c, sl])

    return deg


def _pre_body(cnt_ref, x_ref, dis_ref, xs_ref):
    c = cnt_ref[0, :, 0:1] + cnt_ref[1, :, 0:1] + 1.0
    d = lax.rsqrt(c)
    dis_ref[...] = d
    xs_ref[...] = d * x_ref[...]


def _l1_body(dis_ref, xs_ref, agg_ref, w1_ref, b1_ref, w2_ref, ys2_ref):
    d = dis_ref[...]
    a = d * (agg_ref[0] + agg_ref[1] + xs_ref[...])
    h1 = jnp.maximum(
        jnp.dot(a, w1_ref[...], preferred_element_type=jnp.float32)
        + b1_ref[...], 0.0)
    ys2_ref[...] = d * jnp.dot(h1, w2_ref[...],
                               preferred_element_type=jnp.float32)


def _l2_body(dis_ref, ys2_ref, agg_ref, b2_ref, w3_ref, ys3_ref):
    d = dis_ref[...]
    h2 = jnp.maximum(
        d * (agg_ref[0] + agg_ref[1] + ys2_ref[...]) + b2_ref[...], 0.0)
    ys3_ref[...] = d * jnp.dot(h2, w3_ref[...],
                               preferred_element_type=jnp.float32)


def _l3_body(dis_ref, ys3_ref, agg_ref, b3_ref, out_ref):
    d = dis_ref[...]
    o = d * (agg_ref[0] + agg_ref[1] + ys3_ref[...]) + b3_ref[...]
    lg = o[:, :10]
    m = jnp.max(lg, axis=1, keepdims=True)
    e = jnp.exp(lg - m)
    out_ref[...] = lg - m - jnp.log(jnp.sum(e, axis=1, keepdims=True))


def kernel(x, edge_index, W1, b1, W2, b2, W3, b3):
    N, D_IN = x.shape
    E = edge_index.shape[1]
    H1 = W1.shape[1]
    H2 = W2.shape[1]
    C = W3.shape[1]
    CP = 16

    NPAD = _ceil_to(N + 1, NS * K_EDGE)
    EP = _ceil_to(E, NW * K_EDGE)

    src = edge_index[0]
    dst = edge_index[1]
    if EP != E:
        pad = EP - E
        src = jnp.concatenate([src, jnp.zeros((pad,), src.dtype)])
        dst = jnp.concatenate([dst, jnp.full((pad,), N, dst.dtype)])

    W3p = jnp.pad(W3, ((0, 0), (0, CP - C)))
    b1r = b1.reshape(1, H1)
    b2r = b2.reshape(1, H2)
    b3r = jnp.pad(b3, (0, CP - C)).reshape(1, CP)

    BN = 1000
    G = N // BN
    f32 = jnp.float32

    deg_k = _sc_deg(NPAD, EP)
    cnt = deg_k(dst)

    dis, xs = pl.pallas_call(
        _pre_body,
        grid=(G,),
        in_specs=[
            pl.BlockSpec((NC, BN, 16), lambda i: (0, i, 0)),
            pl.BlockSpec((BN, D_IN), lambda i: (i, 0)),
        ],
        out_specs=[
            pl.BlockSpec((BN, 1), lambda i: (i, 0)),
            pl.BlockSpec((BN, D_IN), lambda i: (i, 0)),
        ],
        out_shape=[
            jax.ShapeDtypeStruct((N, 1), f32),
            jax.ShapeDtypeStruct((N, D_IN), f32),
        ],
    )(cnt, x)

    agg1 = _sc_agg(N, NPAD, EP, D_IN)(xs, src, dst)

    ys2 = pl.pallas_call(
        _l1_body,
        grid=(G,),
        in_specs=[
            pl.BlockSpec((BN, 1), lambda i: (i, 0)),
            pl.BlockSpec((BN, D_IN), lambda i: (i, 0)),
            pl.BlockSpec((NC, BN, D_IN), lambda i: (0, i, 0)),
            pl.BlockSpec((D_IN, H1), lambda i: (0, 0)),
            pl.BlockSpec((1, H1), lambda i: (0, 0)),
            pl.BlockSpec((H1, H2), lambda i: (0, 0)),
        ],
        out_specs=pl.BlockSpec((BN, H2), lambda i: (i, 0)),
        out_shape=jax.ShapeDtypeStruct((N, H2), f32),
    )(dis, xs, agg1, W1, b1r, W2)

    agg2 = _sc_agg(N, NPAD, EP, H2)(ys2, src, dst)

    ys3 = pl.pallas_call(
        _l2_body,
        grid=(G,),
        in_specs=[
            pl.BlockSpec((BN, 1), lambda i: (i, 0)),
            pl.BlockSpec((BN, H2), lambda i: (i, 0)),
            pl.BlockSpec((NC, BN, H2), lambda i: (0, i, 0)),
            pl.BlockSpec((1, H2), lambda i: (0, 0)),
            pl.BlockSpec((H2, CP), lambda i: (0, 0)),
        ],
        out_specs=pl.BlockSpec((BN, CP), lambda i: (i, 0)),
        out_shape=jax.ShapeDtypeStruct((N, CP), f32),
    )(dis, ys2, agg2, b2r, W3p)

    agg3 = _sc_agg(N, NPAD, EP, CP)(ys3, src, dst)

    out = pl.pallas_call(
        _l3_body,
        grid=(G,),
        in_specs=[
            pl.BlockSpec((BN, 1), lambda i: (i, 0)),
            pl.BlockSpec((BN, CP), lambda i: (i, 0)),
            pl.BlockSpec((NC, BN, CP), lambda i: (0, i, 0)),
            pl.BlockSpec((1, CP), lambda i: (0, 0)),
        ],
        out_specs=pl.BlockSpec((BN, C), lambda i: (i, 0)),
        out_shape=jax.ShapeDtypeStruct((N, C), f32),
    )(dis, ys3, agg3, b3r)

    return out


# speedup vs baseline: 15.0049x; 1.0983x over previous
import functools

import jax
import jax.numpy as jnp
from jax import lax
from jax.experimental import pallas as pl
from jax.experimental.pallas import tpu as pltpu
from jax.experimental.pallas import tpu_sc as plsc

NC = 2
NS = 16
NW = NC * NS


def _ceil_to(a, b):
    return (a + b - 1) // b * b


def _sc_agg(N, NPAD, EP, D, K):
    EPT = EP // NW
    CH = EPT // K
    RPS = NPAD // NS
    RCH = [(o, min(K, RPS - o)) for o in range(0, RPS, K)]
    assert EPT % K == 0 and CH % 2 == 0

    mesh = plsc.VectorSubcoreMesh(core_axis_name="c", subcore_axis_name="s",
                                  num_cores=NC, num_subcores=NS)

    @functools.partial(
        pl.kernel,
        out_type=jax.ShapeDtypeStruct((NC, NPAD, D), jnp.float32),
        mesh=mesh,
        scratch_types=[
            pltpu.VMEM((CH, K), jnp.int32),
            pltpu.VMEM((CH, K), jnp.int32),
            pltpu.VMEM((K, D), jnp.float32),
            pltpu.VMEM((K, D), jnp.float32),
            pltpu.VMEM_SHARED((NPAD, D), jnp.float32),
            pltpu.SemaphoreType.DMA,
            pltpu.SemaphoreType.DMA,
        ],
        compiler_params=pltpu.CompilerParams(use_tc_tiling_on_sc=False),
    )
    def agg(table, src2, dst2, out, isrc, idst, r0, r1, acc, g0, g1):
        c = lax.axis_index("c")
        s = lax.axis_index("s")
        w = c * NS + s
        zeros16 = jnp.zeros((16,), jnp.float32)

        def zrow(r, carry):
            for j in range(D // 16):
                r0[r, pl.ds(j * 16, 16)] = zeros16
            return carry

        lax.fori_loop(0, K, zrow, 0)
        sub0 = s * RPS
        for o, n in RCH:
            pltpu.sync_copy(r0.at[pl.ds(0, n)], acc.at[pl.ds(sub0 + o, n)])

        pltpu.sync_copy(src2.at[pl.ds(w * CH, CH)], isrc)
        pltpu.sync_copy(dst2.at[pl.ds(w * CH, CH)], idst)
        plsc.subcore_barrier()

        def gather(i, buf, sem):
            return pltpu.async_copy(table.at[isrc.at[i]], buf, sem)

        def gwait(i, buf, sem):
            pltpu.make_async_copy(table.at[isrc.at[i]], buf, sem).wait()

        def scat(i, buf):
            pltpu.sync_copy(buf, acc.at[idst.at[i]], add=True)

        gather(0, r0, g0)

        def pair(j, carry):
            i = 2 * j
            gather(i + 1, r1, g1)
            gwait(i, r0, g0)
            scat(i, r0)
            gather(i + 2, r0, g0)
            gwait(i + 1, r1, g1)
            scat(i + 1, r1)
            return carry

        lax.fori_loop(0, CH // 2 - 1, pair, 0)
        i = CH - 2
        gather(i + 1, r1, g1)
        gwait(i, r0, g0)
        scat(i, r0)
        gwait(i + 1, r1, g1)
        scat(i + 1, r1)

        plsc.subcore_barrier()
        for o, n in RCH:
            sl = pl.ds(sub0 + o, n)
            pltpu.sync_copy(acc.at[sl], out.at[c, sl])

    return agg


def _sc_deg(NPAD, EP, K):
    D = 16
    EPT = EP // NW
    CH = EPT // K
    RPS = NPAD // NS
    RCH = [(o, min(K, RPS - o)) for o in range(0, RPS, K)]
    FD = 8
    assert EPT % K == 0 and CH % FD == 0

    mesh = plsc.VectorSubcoreMesh(core_axis_name="c", subcore_axis_name="s",
                                  num_cores=NC, num_subcores=NS)

    @functools.partial(
        pl.kernel,
        out_type=jax.ShapeDtypeStruct((NC, NPAD, D), jnp.float32),
        mesh=mesh,
        scratch_types=[
            pltpu.VMEM((CH, K), jnp.int32),
            pltpu.VMEM((K, D), jnp.float32),
            pltpu.VMEM_SHARED((NPAD, D), jnp.float32),
            pltpu.SemaphoreType.DMA,
        ],
        compiler_params=pltpu.CompilerParams(use_tc_tiling_on_sc=False),
    )
    def deg(dst2, out, idst, rows, acc, sem):
        c = lax.axis_index("c")
        s = lax.axis_index("s")
        w = c * NS + s
        zeros16 = jnp.zeros((16,), jnp.float32)
        ones16 = jnp.ones((16,), jnp.float32)

        def fill(vec):
            def body(r, carry):
                rows[r, pl.ds(0, 16)] = vec
                return carry
            lax.fori_loop(0, K, body, 0)

        fill(zeros16)
        sub0 = s * RPS
        for o, n in RCH:
            pltpu.sync_copy(rows.at[pl.ds(0, n)], acc.at[pl.ds(sub0 + o, n)])
        fill(ones16)
        pltpu.sync_copy(dst2.at[pl.ds(w * CH, CH)], idst)
        plsc.subcore_barrier()

        def grp(g, carry):
            for t in range(FD):
                pltpu.async_copy(rows, acc.at[idst.at[g * FD + t]], sem,
                                 add=True)
            for t in range(FD):
                pltpu.make_async_copy(rows, acc.at[idst.at[g * FD + t]],
                                      sem).wait()
            return carry

        lax.fori_loop(0, CH // FD, grp, 0)
        plsc.subcore_barrier()
        for o, n in RCH:
            sl = pl.ds(sub0 + o, n)
            pltpu.sync_copy(acc.at[sl], out.at[c, sl])

    return deg


def _pre_body(cnt_ref, x_ref, dis_ref, xs_ref):
    c = cnt_ref[0, :, 0:1] + cnt_ref[1, :, 0:1] + 1.0
    d = lax.rsqrt(c)
    dis_ref[...] = d
    xs_ref[...] = d * x_ref[...]


def _l1_body(dis_ref, xs_ref, agg_ref, w1_ref, b1_ref, w2_ref, ys2_ref):
    d = dis_ref[...]
    a = d * (agg_ref[0] + agg_ref[1] + xs_ref[...])
    h1 = jnp.maximum(
        jnp.dot(a, w1_ref[...], preferred_element_type=jnp.float32)
        + b1_ref[...], 0.0)
    ys2_ref[...] = d * jnp.dot(h1, w2_ref[...],
                               preferred_element_type=jnp.float32)


def _l2_body(dis_ref, ys2_ref, agg_ref, b2_ref, w3_ref, ys3_ref):
    d = dis_ref[...]
    h2 = jnp.maximum(
        d * (agg_ref[0] + agg_ref[1] + ys2_ref[...]) + b2_ref[...], 0.0)
    ys3_ref[...] = d * jnp.dot(h2, w3_ref[...],
                               preferred_element_type=jnp.float32)


def _l3_body(dis_ref, ys3_ref, agg_ref, b3_ref, out_ref):
    d = dis_ref[...]
    o = d * (agg_ref[0] + agg_ref[1] + ys3_ref[...]) + b3_ref[...]
    lg = o[:, :10]
    m = jnp.max(lg, axis=1, keepdims=True)
    e = jnp.exp(lg - m)
    out_ref[...] = lg - m - jnp.log(jnp.sum(e, axis=1, keepdims=True))


def kernel(x, edge_index, W1, b1, W2, b2, W3, b3):
    N, D_IN = x.shape
    E = edge_index.shape[1]
    H1 = W1.shape[1]
    H2 = W2.shape[1]
    C = W3.shape[1]
    CP = 16

    K1 = 96
    K2 = 128
    NPAD = _ceil_to(N + 1, NS * 8)

    src = edge_index[0]
    dst = edge_index[1]

    def _chunked(K):
        EP = _ceil_to(E, 2 * NW * K)
        s, d = src, dst
        if EP != E:
            pad = EP - E
            s = jnp.concatenate([s, jnp.zeros((pad,), s.dtype)])
            d = jnp.concatenate([d, jnp.full((pad,), N, d.dtype)])
        return EP, s.reshape(EP // K, K), d.reshape(EP // K, K)

    EP1, src2a, dst2a = _chunked(K1)
    EP2, src2b, dst2b = _chunked(K2)

    W3p = jnp.pad(W3, ((0, 0), (0, CP - C)))
    b1r = b1.reshape(1, H1)
    b2r = b2.reshape(1, H2)
    b3r = jnp.pad(b3, (0, CP - C)).reshape(1, CP)

    BN = 1000
    G = N // BN
    f32 = jnp.float32

    cnt = _sc_deg(NPAD, EP2, K2)(dst2b)

    dis, xs = pl.pallas_call(
        _pre_body,
        grid=(G,),
        in_specs=[
            pl.BlockSpec((NC, BN, 16), lambda i: (0, i, 0)),
            pl.BlockSpec((BN, D_IN), lambda i: (i, 0)),
        ],
        out_specs=[
            pl.BlockSpec((BN, 1), lambda i: (i, 0)),
            pl.BlockSpec((BN, D_IN), lambda i: (i, 0)),
        ],
        out_shape=[
            jax.ShapeDtypeStruct((N, 1), f32),
            jax.ShapeDtypeStruct((N, D_IN), f32),
        ],
    )(cnt, x)

    agg1 = _sc_agg(N, NPAD, EP1, D_IN, K1)(xs, src2a, dst2a)

    ys2 = pl.pallas_call(
        _l1_body,
        grid=(G,),
        in_specs=[
            pl.BlockSpec((BN, 1), lambda i: (i, 0)),
            pl.BlockSpec((BN, D_IN), lambda i: (i, 0)),
            pl.BlockSpec((NC, BN, D_IN), lambda i: (0, i, 0)),
            pl.BlockSpec((D_IN, H1), lambda i: (0, 0)),
            pl.BlockSpec((1, H1), lambda i: (0, 0)),
            pl.BlockSpec((H1, H2), lambda i: (0, 0)),
        ],
        out_specs=pl.BlockSpec((BN, H2), lambda i: (i, 0)),
        out_shape=jax.ShapeDtypeStruct((N, H2), f32),
    )(dis, xs, agg1, W1, b1r, W2)

    agg2 = _sc_agg(N, NPAD, EP2, H2, K2)(ys2, src2b, dst2b)

    ys3 = pl.pallas_call(
        _l2_body,
        grid=(G,),
        in_specs=[
            pl.BlockSpec((BN, 1), lambda i: (i, 0)),
            pl.BlockSpec((BN, H2), lambda i: (i, 0)),
            pl.BlockSpec((NC, BN, H2), lambda i: (0, i, 0)),
            pl.BlockSpec((1, H2), lambda i: (0, 0)),
            pl.BlockSpec((H2, CP), lambda i: (0, 0)),
        ],
        out_specs=pl.BlockSpec((BN, CP), lambda i: (i, 0)),
        out_shape=jax.ShapeDtypeStruct((N, CP), f32),
    )(dis, ys2, agg2, b2r, W3p)

    agg3 = _sc_agg(N, NPAD, EP2, CP, K2)(ys3, src2b, dst2b)

    out = pl.pallas_call(
        _l3_body,
        grid=(G,),
        in_specs=[
            pl.BlockSpec((BN, 1), lambda i: (i, 0)),
            pl.BlockSpec((BN, CP), lambda i: (i, 0)),
            pl.BlockSpec((NC, BN, CP), lambda i: (0, i, 0)),
            pl.BlockSpec((1, CP), lambda i: (0, 0)),
        ],
        out_specs=pl.BlockSpec((BN, C), lambda i: (i, 0)),
        out_shape=jax.ShapeDtypeStruct((N, C), f32),
    )(dis, ys3, agg3, b3r)

    return out


# speedup vs baseline: 25.1975x; 1.6793x over previous
import functools

import jax
import jax.numpy as jnp
from jax import lax
from jax.experimental import pallas as pl
from jax.experimental.pallas import tpu as pltpu
from jax.experimental.pallas import tpu_sc as plsc

NC = 2
NS = 16
NW = NC * NS


def _ceil_to(a, b):
    return (a + b - 1) // b * b


def _sc_agg(N, NPAD, CH0, CH1, D, K):
    RPS = NPAD // NS
    RCH = [(o, min(K, RPS - o)) for o in range(0, RPS, K)]
    CHM = max(CH0, CH1)
    assert CH0 % 2 == 0 and CH1 % 2 == 0 and CH0 >= 2 and CH1 >= 2

    mesh = plsc.VectorSubcoreMesh(core_axis_name="c", subcore_axis_name="s",
                                  num_cores=NC, num_subcores=NS)

    @functools.partial(
        pl.kernel,
        out_type=jax.ShapeDtypeStruct((NC, NPAD, D), jnp.float32),
        mesh=mesh,
        scratch_types=[
            pltpu.VMEM((CHM, K), jnp.int32),
            pltpu.VMEM((CHM, K), jnp.int32),
            pltpu.VMEM((K, D), jnp.float32),
            pltpu.VMEM((K, D), jnp.float32),
            pltpu.VMEM_SHARED((NPAD, D), jnp.float32),
            pltpu.SemaphoreType.DMA,
            pltpu.SemaphoreType.DMA,
        ],
        compiler_params=pltpu.CompilerParams(use_tc_tiling_on_sc=False),
    )
    def agg(table, src2, dst2, out, isrc, idst, r0, r1, acc, g0, g1):
        c = lax.axis_index("c")
        s = lax.axis_index("s")
        ch = jnp.where(c == 0, CH0, CH1)
        zeros16 = jnp.zeros((16,), jnp.float32)

        def zrow(r, carry):
            for j in range(D // 16):
                r0[r, pl.ds(j * 16, 16)] = zeros16
            return carry

        lax.fori_loop(0, K, zrow, 0)
        sub0 = s * RPS
        for o, n in RCH:
            pltpu.sync_copy(r0.at[pl.ds(0, n)], acc.at[pl.ds(sub0 + o, n)])

        @pl.when(c == 0)
        def _():
            pltpu.sync_copy(src2.at[pl.ds(s * CH0, CH0)],
                            isrc.at[pl.ds(0, CH0)])
            pltpu.sync_copy(dst2.at[pl.ds(s * CH0, CH0)],
                            idst.at[pl.ds(0, CH0)])

        @pl.when(c == 1)
        def _():
            pltpu.sync_copy(src2.at[pl.ds(NS * CH0 + s * CH1, CH1)],
                            isrc.at[pl.ds(0, CH1)])
            pltpu.sync_copy(dst2.at[pl.ds(NS * CH0 + s * CH1, CH1)],
                            idst.at[pl.ds(0, CH1)])

        plsc.subcore_barrier()

        def gather(i, buf, sem):
            return pltpu.async_copy(table.at[isrc.at[i]], buf, sem)

        def gwait(i, buf, sem):
            pltpu.make_async_copy(table.at[isrc.at[i]], buf, sem).wait()

        def scat(i, buf):
            pltpu.sync_copy(buf, acc.at[idst.at[i]], add=True)

        gather(0, r0, g0)

        def pair(j, carry):
            i = 2 * j

            @pl.when(i + 1 < ch)
            def _():
                gather(i + 1, r1, g1)

            @pl.when(i < ch)
            def _():
                gwait(i, r0, g0)
                scat(i, r0)

            @pl.when(i + 2 < ch)
            def _():
                gather(i + 2, r0, g0)

            @pl.when(i + 1 < ch)
            def _():
                gwait(i + 1, r1, g1)
                scat(i + 1, r1)

            return carry

        lax.fori_loop(0, CHM // 2, pair, 0)

        plsc.subcore_barrier()
        for o, n in RCH:
            sl = pl.ds(sub0 + o, n)
            pltpu.sync_copy(acc.at[sl], out.at[c, sl])

    return agg


def _sc_deg(NPAD, CH0, CH1, K):
    D = 16
    RPS = NPAD // NS
    RCH = [(o, min(K, RPS - o)) for o in range(0, RPS, K)]
    CHM = max(CH0, CH1)
    FD = 8

    mesh = plsc.VectorSubcoreMesh(core_axis_name="c", subcore_axis_name="s",
                                  num_cores=NC, num_subcores=NS)

    @functools.partial(
        pl.kernel,
        out_type=jax.ShapeDtypeStruct((NC, NPAD, D), jnp.float32),
        mesh=mesh,
        scratch_types=[
            pltpu.VMEM((CHM, K), jnp.int32),
            pltpu.VMEM((K, D), jnp.float32),
            pltpu.VMEM_SHARED((NPAD, D), jnp.float32),
            pltpu.SemaphoreType.DMA,
        ],
        compiler_params=pltpu.CompilerParams(use_tc_tiling_on_sc=False),
    )
    def deg(dst2, out, idst, rows, acc, sem):
        c = lax.axis_index("c")
        s = lax.axis_index("s")
        ch = jnp.where(c == 0, CH0, CH1)
        zeros16 = jnp.zeros((16,), jnp.float32)
        ones16 = jnp.ones((16,), jnp.float32)

        def fill(vec):
            def body(r, carry):
                rows[r, pl.ds(0, 16)] = vec
                return carry
            lax.fori_loop(0, K, body, 0)

        fill(zeros16)
        sub0 = s * RPS
        for o, n in RCH:
            pltpu.sync_copy(rows.at[pl.ds(0, n)], acc.at[pl.ds(sub0 + o, n)])
        fill(ones16)

        @pl.when(c == 0)
        def _():
            pltpu.sync_copy(dst2.at[pl.ds(s * CH0, CH0)],
                            idst.at[pl.ds(0, CH0)])

        @pl.when(c == 1)
        def _():
            pltpu.sync_copy(dst2.at[pl.ds(NS * CH0 + s * CH1, CH1)],
                            idst.at[pl.ds(0, CH1)])

        plsc.subcore_barrier()

        def grp(g, carry):
            for t in range(FD):
                i = g * FD + t

                @pl.when(i < ch)
                def _():
                    pltpu.async_copy(rows, acc.at[idst.at[i]], sem, add=True)

            for t in range(FD):
                i = g * FD + t

                @pl.when(i < ch)
                def _():
                    pltpu.make_async_copy(rows, acc.at[idst.at[i]],
                                          sem).wait()

            return carry

        lax.fori_loop(0, (CHM + FD - 1) // FD, grp, 0)
        plsc.subcore_barrier()
        for o, n in RCH:
            sl = pl.ds(sub0 + o, n)
            pltpu.sync_copy(acc.at[sl], out.at[c, sl])

    return deg


def _pre_body(cnt_ref, x_ref, dis_ref, xs_ref):
    c = cnt_ref[0, :, 0:1] + cnt_ref[1, :, 0:1] + 1.0
    d = lax.rsqrt(c)
    dis_ref[...] = d
    xs_ref[...] = d * x_ref[...]


def _l1_body(dis_ref, xs_ref, agg_ref, w1_ref, b1_ref, w2_ref, ys2_ref):
    d = dis_ref[...]
    a = d * (agg_ref[0] + agg_ref[1] + xs_ref[...])
    h1 = jnp.maximum(
        jnp.dot(a, w1_ref[...], preferred_element_type=jnp.float32)
        + b1_ref[...], 0.0)
    ys2_ref[...] = d * jnp.dot(h1, w2_ref[...],
                               preferred_element_type=jnp.float32)


def _l2_body(dis_ref, ys2_ref, agg_ref, b2_ref, w3_ref, ys3_ref):
    d = dis_ref[...]
    h2 = jnp.maximum(
        d * (agg_ref[0] + agg_ref[1] + ys2_ref[...]) + b2_ref[...], 0.0)
    ys3_ref[...] = d * jnp.dot(h2, w3_ref[...],
                               preferred_element_type=jnp.float32)


def _l3_body(dis_ref, ys3_ref, agg_ref, b3_ref, out_ref):
    d = dis_ref[...]
    o = d * (agg_ref[0] + agg_ref[1] + ys3_ref[...]) + b3_ref[...]
    lg = o[:, :10]
    m = jnp.max(lg, axis=1, keepdims=True)
    e = jnp.exp(lg - m)
    out_ref[...] = lg - m - jnp.log(jnp.sum(e, axis=1, keepdims=True))


def kernel(x, edge_index, W1, b1, W2, b2, W3, b3):
    N, D_IN = x.shape
    E = edge_index.shape[1]
    H1 = W1.shape[1]
    H2 = W2.shape[1]
    C = W3.shape[1]
    CP = 16

    K1 = 64
    K2 = 128
    F0 = 0.75
    NPAD = _ceil_to(N + 1, NS * 8)

    src = edge_index[0]
    dst = edge_index[1]

    def _chunked(K):
        tot = -(-E // (NS * K))
        ch0 = max(2, int(round(F0 * tot / 2)) * 2)
        ch1 = max(2, -(-(tot - ch0) // 2) * 2)
        ep = NS * K * (ch0 + ch1)
        s, d = src, dst
        if ep != E:
            pad = ep - E
            s = jnp.concatenate([s, jnp.zeros((pad,), s.dtype)])
            d = jnp.concatenate([d, jnp.full((pad,), N, d.dtype)])
        return ch0, ch1, s.reshape(ep // K, K), d.reshape(ep // K, K)

    A0, A1, src2a, dst2a = _chunked(K1)
    B0, B1, src2b, dst2b = _chunked(K2)

    W3p = jnp.pad(W3, ((0, 0), (0, CP - C)))
    b1r = b1.reshape(1, H1)
    b2r = b2.reshape(1, H2)
    b3r = jnp.pad(b3, (0, CP - C)).reshape(1, CP)

    BN = 1000
    G = N // BN
    f32 = jnp.float32

    cnt = _sc_deg(NPAD, B0, B1, K2)(dst2b)

    dis, xs = pl.pallas_call(
        _pre_body,
        grid=(G,),
        in_specs=[
            pl.BlockSpec((NC, BN, 16), lambda i: (0, i, 0)),
            pl.BlockSpec((BN, D_IN), lambda i: (i, 0)),
        ],
        out_specs=[
            pl.BlockSpec((BN, 1), lambda i: (i, 0)),
            pl.BlockSpec((BN, D_IN), lambda i: (i, 0)),
        ],
        out_shape=[
            jax.ShapeDtypeStruct((N, 1), f32),
            jax.ShapeDtypeStruct((N, D_IN), f32),
        ],
    )(cnt, x)

    agg1 = _sc_agg(N, NPAD, A0, A1, D_IN, K1)(xs, src2a, dst2a)

    ys2 = pl.pallas_call(
        _l1_body,
        grid=(G,),
        in_specs=[
            pl.BlockSpec((BN, 1), lambda i: (i, 0)),
            pl.BlockSpec((BN, D_IN), lambda i: (i, 0)),
            pl.BlockSpec((NC, BN, D_IN), lambda i: (0, i, 0)),
            pl.BlockSpec((D_IN, H1), lambda i: (0, 0)),
            pl.BlockSpec((1, H1), lambda i: (0, 0)),
            pl.BlockSpec((H1, H2), lambda i: (0, 0)),
        ],
        out_specs=pl.BlockSpec((BN, H2), lambda i: (i, 0)),
        out_shape=jax.ShapeDtypeStruct((N, H2), f32),
    )(dis, xs, agg1, W1, b1r, W2)

    agg2 = _sc_agg(N, NPAD, B0, B1, H2, K2)(ys2, src2b, dst2b)

    ys3 = pl.pallas_call(
        _l2_body,
        grid=(G,),
        in_specs=[
            pl.BlockSpec((BN, 1), lambda i: (i, 0)),
            pl.BlockSpec((BN, H2), lambda i: (i, 0)),
            pl.BlockSpec((NC, BN, H2), lambda i: (0, i, 0)),
            pl.BlockSpec((1, H2), lambda i: (0, 0)),
            pl.BlockSpec((H2, CP), lambda i: (0, 0)),
        ],
        out_specs=pl.BlockSpec((BN, CP), lambda i: (i, 0)),
        out_shape=jax.ShapeDtypeStruct((N, CP), f32),
    )(dis, ys2, agg2, b2r, W3p)

    agg3 = _sc_agg(N, NPAD, B0, B1, CP, K2)(ys3, src2b, dst2b)

    out = pl.pallas_call(
        _l3_body,
        grid=(G,),
        in_specs=[
            pl.BlockSpec((BN, 1), lambda i: (i, 0)),
            pl.BlockSpec((BN, CP), lambda i: (i, 0)),
            pl.BlockSpec((NC, BN, CP), lambda i: (0, i, 0)),
            pl.BlockSpec((1, CP), lambda i: (0, 0)),
        ],
        out_specs=pl.BlockSpec((BN, C), lambda i: (i, 0)),
        out_shape=jax.ShapeDtypeStruct((N, C), f32),
    )(dis, ys3, agg3, b3r)

    return out


# speedup vs baseline: 26.7481x; 1.0615x over previous
import functools

import jax
import jax.numpy as jnp
from jax import lax
from jax.experimental import pallas as pl
from jax.experimental.pallas import tpu as pltpu
from jax.experimental.pallas import tpu_sc as plsc

NC = 2
NS = 16
NW = NC * NS


def _ceil_to(a, b):
    return (a + b - 1) // b * b


def _sc_agg(N, NPAD, CH0, CH1, D, K, NB):
    RPS = NPAD // NS
    RCH = [(o, min(K, RPS - o)) for o in range(0, RPS, K)]
    CHM = max(CH0, CH1)
    LAG = max(1, NB // 2)
    VMAX = _ceil_to(CHM + NB, NB)

    mesh = plsc.VectorSubcoreMesh(core_axis_name="c", subcore_axis_name="s",
                                  num_cores=NC, num_subcores=NS)

    @functools.partial(
        pl.kernel,
        out_type=jax.ShapeDtypeStruct((NC, NPAD, D), jnp.float32),
        mesh=mesh,
        scratch_types=(
            [pltpu.VMEM((CHM, K), jnp.int32),
             pltpu.VMEM((CHM, K), jnp.int32)]
            + [pltpu.VMEM((K, D), jnp.float32) for _ in range(NB)]
            + [pltpu.SemaphoreType.DMA for _ in range(2 * NB)]
            + [pltpu.VMEM_SHARED((NPAD, D), jnp.float32)]
        ),
        compiler_params=pltpu.CompilerParams(use_tc_tiling_on_sc=False),
    )
    def agg(table, src2, dst2, out, *scr):
        isrc, idst = scr[0], scr[1]
        bufs = scr[2:2 + NB]
        gsem = scr[2 + NB:2 + 2 * NB]
        ssem = scr[2 + 2 * NB:2 + 3 * NB]
        acc = scr[2 + 3 * NB]
        c = lax.axis_index("c")
        s = lax.axis_index("s")
        ch = jnp.where(c == 0, CH0, CH1)
        zeros16 = jnp.zeros((16,), jnp.float32)
        r0 = bufs[0]

        def zrow(r, carry):
            for j in range(D // 16):
                r0[r, pl.ds(j * 16, 16)] = zeros16
            return carry

        lax.fori_loop(0, K, zrow, 0)
        sub0 = s * RPS
        for o, n in RCH:
            pltpu.sync_copy(r0.at[pl.ds(0, n)], acc.at[pl.ds(sub0 + o, n)])

        @pl.when(c == 0)
        def _():
            pltpu.sync_copy(src2.at[pl.ds(s * CH0, CH0)],
                            isrc.at[pl.ds(0, CH0)])
            pltpu.sync_copy(dst2.at[pl.ds(s * CH0, CH0)],
                            idst.at[pl.ds(0, CH0)])

        @pl.when(c == 1)
        def _():
            pltpu.sync_copy(src2.at[pl.ds(NS * CH0 + s * CH1, CH1)],
                            isrc.at[pl.ds(0, CH1)])
            pltpu.sync_copy(dst2.at[pl.ds(NS * CH0 + s * CH1, CH1)],
                            idst.at[pl.ds(0, CH1)])

        plsc.subcore_barrier()

        def gstart(i, b):
            pltpu.async_copy(table.at[isrc.at[i]], bufs[b], gsem[b])

        def gwait(i, b):
            pltpu.make_async_copy(table.at[isrc.at[i]], bufs[b],
                                  gsem[b]).wait()

        def sstart(i, b):
            pltpu.async_copy(bufs[b], acc.at[idst.at[i]], ssem[b], add=True)

        def swait(i, b):
            pltpu.make_async_copy(bufs[b], acc.at[idst.at[i]],
                                  ssem[b]).wait()

        def step(j, carry):
            for b in range(NB):
                v = j * NB + b

                @pl.when((v >= NB) & (v - NB < ch))
                def _():
                    swait(v - NB, b)

                @pl.when(v < ch)
                def _():
                    gstart(v, b)

                u = v - LAG
                bu = (b - LAG) % NB

                @pl.when((u >= 0) & (u < ch))
                def _():
                    gwait(u, bu)
                    sstart(u, bu)

            return carry

        lax.fori_loop(0, VMAX // NB, step, 0)

        plsc.subcore_barrier()
        for o, n in RCH:
            sl = pl.ds(sub0 + o, n)
            pltpu.sync_copy(acc.at[sl], out.at[c, sl])

    return agg


def _sc_deg(NPAD, CH0, CH1, K):
    D = 16
    RPS = NPAD // NS
    RCH = [(o, min(K, RPS - o)) for o in range(0, RPS, K)]
    CHM = max(CH0, CH1)
    FD = 8

    mesh = plsc.VectorSubcoreMesh(core_axis_name="c", subcore_axis_name="s",
                                  num_cores=NC, num_subcores=NS)

    @functools.partial(
        pl.kernel,
        out_type=jax.ShapeDtypeStruct((NC, NPAD, D), jnp.float32),
        mesh=mesh,
        scratch_types=[
            pltpu.VMEM((CHM, K), jnp.int32),
            pltpu.VMEM((K, D), jnp.float32),
            pltpu.VMEM_SHARED((NPAD, D), jnp.float32),
            pltpu.SemaphoreType.DMA,
        ],
        compiler_params=pltpu.CompilerParams(use_tc_tiling_on_sc=False),
    )
    def deg(dst2, out, idst, rows, acc, sem):
        c = lax.axis_index("c")
        s = lax.axis_index("s")
        ch = jnp.where(c == 0, CH0, CH1)
        zeros16 = jnp.zeros((16,), jnp.float32)
        ones16 = jnp.ones((16,), jnp.float32)

        def fill(vec):
            def body(r, carry):
                rows[r, pl.ds(0, 16)] = vec
                return carry
            lax.fori_loop(0, K, body, 0)

        fill(zeros16)
        sub0 = s * RPS
        for o, n in RCH:
            pltpu.sync_copy(rows.at[pl.ds(0, n)], acc.at[pl.ds(sub0 + o, n)])
        fill(ones16)

        @pl.when(c == 0)
        def _():
            pltpu.sync_copy(dst2.at[pl.ds(s * CH0, CH0)],
                            idst.at[pl.ds(0, CH0)])

        @pl.when(c == 1)
        def _():
            pltpu.sync_copy(dst2.at[pl.ds(NS * CH0 + s * CH1, CH1)],
                            idst.at[pl.ds(0, CH1)])

        plsc.subcore_barrier()

        def grp(g, carry):
            for t in range(FD):
                i = g * FD + t

                @pl.when(i < ch)
                def _():
                    pltpu.async_copy(rows, acc.at[idst.at[i]], sem, add=True)

            for t in range(FD):
                i = g * FD + t

                @pl.when(i < ch)
                def _():
                    pltpu.make_async_copy(rows, acc.at[idst.at[i]],
                                          sem).wait()

            return carry

        lax.fori_loop(0, (CHM + FD - 1) // FD, grp, 0)
        plsc.subcore_barrier()
        for o, n in RCH:
            sl = pl.ds(sub0 + o, n)
            pltpu.sync_copy(acc.at[sl], out.at[c, sl])

    return deg


def _pre_body(cnt_ref, x_ref, dis_ref, xs_ref):
    c = cnt_ref[0, :, 0:1] + cnt_ref[1, :, 0:1] + 1.0
    d = lax.rsqrt(c)
    dis_ref[...] = d
    xs_ref[...] = d * x_ref[...]


def _l1_body(dis_ref, xs_ref, agg_ref, w1_ref, b1_ref, w2_ref, ys2_ref):
    d = dis_ref[...]
    a = d * (agg_ref[0] + agg_ref[1] + xs_ref[...])
    h1 = jnp.maximum(
        jnp.dot(a, w1_ref[...], preferred_element_type=jnp.float32)
        + b1_ref[...], 0.0)
    ys2_ref[...] = d * jnp.dot(h1, w2_ref[...],
                               preferred_element_type=jnp.float32)


def _l2_body(dis_ref, ys2_ref, agg_ref, b2_ref, w3_ref, ys3_ref):
    d = dis_ref[...]
    h2 = jnp.maximum(
        d * (agg_ref[0] + agg_ref[1] + ys2_ref[...]) + b2_ref[...], 0.0)
    ys3_ref[...] = d * jnp.dot(h2, w3_ref[...],
                               preferred_element_type=jnp.float32)


def _l3_body(dis_ref, ys3_ref, agg_ref, b3_ref, out_ref):
    d = dis_ref[...]
    o = d * (agg_ref[0] + agg_ref[1] + ys3_ref[...]) + b3_ref[...]
    lg = o[:, :10]
    m = jnp.max(lg, axis=1, keepdims=True)
    e = jnp.exp(lg - m)
    out_ref[...] = lg - m - jnp.log(jnp.sum(e, axis=1, keepdims=True))


def kernel(x, edge_index, W1, b1, W2, b2, W3, b3):
    N, D_IN = x.shape
    E = edge_index.shape[1]
    H1 = W1.shape[1]
    H2 = W2.shape[1]
    C = W3.shape[1]
    CP = 16

    K1 = 64
    K2 = 128
    F0 = 0.75
    NPAD = _ceil_to(N + 1, NS * 8)

    src = edge_index[0]
    dst = edge_index[1]

    def _chunked(K):
        tot = -(-E // (NS * K))
        ch0 = max(2, int(round(F0 * tot / 2)) * 2)
        ch1 = max(2, -(-(tot - ch0) // 2) * 2)
        ep = NS * K * (ch0 + ch1)
        s, d = src, dst
        if ep != E:
            pad = ep - E
            s = jnp.concatenate([s, jnp.zeros((pad,), s.dtype)])
            d = jnp.concatenate([d, jnp.full((pad,), N, d.dtype)])
        return ch0, ch1, s.reshape(ep // K, K), d.reshape(ep // K, K)

    A0, A1, src2a, dst2a = _chunked(K1)
    B0, B1, src2b, dst2b = _chunked(K2)

    W3p = jnp.pad(W3, ((0, 0), (0, CP - C)))
    b1r = b1.reshape(1, H1)
    b2r = b2.reshape(1, H2)
    b3r = jnp.pad(b3, (0, CP - C)).reshape(1, CP)

    BN = 1000
    G = N // BN
    f32 = jnp.float32

    cnt = _sc_deg(NPAD, B0, B1, K2)(dst2b)

    dis, xs = pl.pallas_call(
        _pre_body,
        grid=(G,),
        in_specs=[
            pl.BlockSpec((NC, BN, 16), lambda i: (0, i, 0)),
            pl.BlockSpec((BN, D_IN), lambda i: (i, 0)),
        ],
        out_specs=[
            pl.BlockSpec((BN, 1), lambda i: (i, 0)),
            pl.BlockSpec((BN, D_IN), lambda i: (i, 0)),
        ],
        out_shape=[
            jax.ShapeDtypeStruct((N, 1), f32),
            jax.ShapeDtypeStruct((N, D_IN), f32),
        ],
    )(cnt, x)

    agg1 = _sc_agg(N, NPAD, A0, A1, D_IN, K1, 2)(xs, src2a, dst2a)

    ys2 = pl.pallas_call(
        _l1_body,
        grid=(G,),
        in_specs=[
            pl.BlockSpec((BN, 1), lambda i: (i, 0)),
            pl.BlockSpec((BN, D_IN), lambda i: (i, 0)),
            pl.BlockSpec((NC, BN, D_IN), lambda i: (0, i, 0)),
            pl.BlockSpec((D_IN, H1), lambda i: (0, 0)),
            pl.BlockSpec((1, H1), lambda i: (0, 0)),
            pl.BlockSpec((H1, H2), lambda i: (0, 0)),
        ],
        out_specs=pl.BlockSpec((BN, H2), lambda i: (i, 0)),
        out_shape=jax.ShapeDtypeStruct((N, H2), f32),
    )(dis, xs, agg1, W1, b1r, W2)

    agg2 = _sc_agg(N, NPAD, B0, B1, H2, K2, 6)(ys2, src2b, dst2b)

    ys3 = pl.pallas_call(
        _l2_body,
        grid=(G,),
        in_specs=[
            pl.BlockSpec((BN, 1), lambda i: (i, 0)),
            pl.BlockSpec((BN, H2), lambda i: (i, 0)),
            pl.BlockSpec((NC, BN, H2), lambda i: (0, i, 0)),
            pl.BlockSpec((1, H2), lambda i: (0, 0)),
            pl.BlockSpec((H2, CP), lambda i: (0, 0)),
        ],
        out_specs=pl.BlockSpec((BN, CP), lambda i: (i, 0)),
        out_shape=jax.ShapeDtypeStruct((N, CP), f32),
    )(dis, ys2, agg2, b2r, W3p)

    agg3 = _sc_agg(N, NPAD, B0, B1, CP, K2, 6)(ys3, src2b, dst2b)

    out = pl.pallas_call(
        _l3_body,
        grid=(G,),
        in_specs=[
            pl.BlockSpec((BN, 1), lambda i: (i, 0)),
            pl.BlockSpec((BN, CP), lambda i: (i, 0)),
            pl.BlockSpec((NC, BN, CP), lambda i: (0, i, 0)),
            pl.BlockSpec((1, CP), lambda i: (0, 0)),
        ],
        out_specs=pl.BlockSpec((BN, C), lambda i: (i, 0)),
        out_shape=jax.ShapeDtypeStruct((N, C), f32),
    )(dis, ys3, agg3, b3r)

    return out


# speedup vs baseline: 29.1883x; 1.0912x over previous
import functools

import jax
import jax.numpy as jnp
from jax import lax
from jax.experimental import pallas as pl
from jax.experimental.pallas import tpu as pltpu
from jax.experimental.pallas import tpu_sc as plsc

NC = 2
NS = 16
NW = NC * NS


def _ceil_to(a, b):
    return (a + b - 1) // b * b


def _sc_agg(N, NPAD, CH0, CH1, D, K, NB):
    RPS = NPAD // NS
    RCH = [(o, min(K, RPS - o)) for o in range(0, RPS, K)]
    CHM = max(CH0, CH1)
    LAG = max(1, NB // 2)
    VMAX = _ceil_to(CHM + NB, NB)

    mesh = plsc.VectorSubcoreMesh(core_axis_name="c", subcore_axis_name="s",
                                  num_cores=NC, num_subcores=NS)

    @functools.partial(
        pl.kernel,
        out_type=jax.ShapeDtypeStruct((NC, NPAD, D), jnp.float32),
        mesh=mesh,
        scratch_types=(
            [pltpu.VMEM((CHM, K), jnp.int32),
             pltpu.VMEM((CHM, K), jnp.int32)]
            + [pltpu.VMEM((K, D), jnp.float32) for _ in range(NB)]
            + [pltpu.SemaphoreType.DMA for _ in range(2 * NB)]
            + [pltpu.VMEM_SHARED((NPAD, D), jnp.float32)]
        ),
        compiler_params=pltpu.CompilerParams(use_tc_tiling_on_sc=False),
    )
    def agg(table, src2, dst2, out, *scr):
        isrc, idst = scr[0], scr[1]
        bufs = scr[2:2 + NB]
        gsem = scr[2 + NB:2 + 2 * NB]
        ssem = scr[2 + 2 * NB:2 + 3 * NB]
        acc = scr[2 + 3 * NB]
        c = lax.axis_index("c")
        s = lax.axis_index("s")
        ch = jnp.where(c == 0, CH0, CH1)
        zeros16 = jnp.zeros((16,), jnp.float32)
        r0 = bufs[0]

        def zrow(r, carry):
            for j in range(D // 16):
                r0[r, pl.ds(j * 16, 16)] = zeros16
            return carry

        lax.fori_loop(0, K, zrow, 0)
        sub0 = s * RPS
        for o, n in RCH:
            pltpu.sync_copy(r0.at[pl.ds(0, n)], acc.at[pl.ds(sub0 + o, n)])

        @pl.when(c == 0)
        def _():
            pltpu.sync_copy(src2.at[pl.ds(s * CH0, CH0)],
                            isrc.at[pl.ds(0, CH0)])
            pltpu.sync_copy(dst2.at[pl.ds(s * CH0, CH0)],
                            idst.at[pl.ds(0, CH0)])

        @pl.when(c == 1)
        def _():
            pltpu.sync_copy(src2.at[pl.ds(NS * CH0 + s * CH1, CH1)],
                            isrc.at[pl.ds(0, CH1)])
            pltpu.sync_copy(dst2.at[pl.ds(NS * CH0 + s * CH1, CH1)],
                            idst.at[pl.ds(0, CH1)])

        plsc.subcore_barrier()

        def gstart(i, b):
            pltpu.async_copy(table.at[isrc.at[i]], bufs[b], gsem[b])

        def gwait(i, b):
            pltpu.make_async_copy(table.at[isrc.at[i]], bufs[b],
                                  gsem[b]).wait()

        def sstart(i, b):
            pltpu.async_copy(bufs[b], acc.at[idst.at[i]], ssem[b], add=True)

        def swait(i, b):
            pltpu.make_async_copy(bufs[b], acc.at[idst.at[i]],
                                  ssem[b]).wait()

        def step(j, carry):
            for b in range(NB):
                v = j * NB + b

                @pl.when((v >= NB) & (v - NB < ch))
                def _():
                    swait(v - NB, b)

                @pl.when(v < ch)
                def _():
                    gstart(v, b)

                u = v - LAG
                bu = (b - LAG) % NB

                @pl.when((u >= 0) & (u < ch))
                def _():
                    gwait(u, bu)
                    sstart(u, bu)

            return carry

        lax.fori_loop(0, VMAX // NB, step, 0)

        plsc.subcore_barrier()
        for o, n in RCH:
            sl = pl.ds(sub0 + o, n)
            pltpu.sync_copy(acc.at[sl], out.at[c, sl])

    return agg


def _sc_deg(NPAD, CH0, CH1, K):
    D = 16
    RPS = NPAD // NS
    RCH = [(o, min(K, RPS - o)) for o in range(0, RPS, K)]
    CHM = max(CH0, CH1)
    FD = 8

    mesh = plsc.VectorSubcoreMesh(core_axis_name="c", subcore_axis_name="s",
                                  num_cores=NC, num_subcores=NS)

    @functools.partial(
        pl.kernel,
        out_type=jax.ShapeDtypeStruct((NC, NPAD, D), jnp.float32),
        mesh=mesh,
        scratch_types=[
            pltpu.VMEM((CHM, K), jnp.int32),
            pltpu.VMEM((K, D), jnp.float32),
            pltpu.VMEM_SHARED((NPAD, D), jnp.float32),
            pltpu.SemaphoreType.DMA,
        ],
        compiler_params=pltpu.CompilerParams(use_tc_tiling_on_sc=False),
    )
    def deg(dst2, out, idst, rows, acc, sem):
        c = lax.axis_index("c")
        s = lax.axis_index("s")
        ch = jnp.where(c == 0, CH0, CH1)
        zeros16 = jnp.zeros((16,), jnp.float32)
        ones16 = jnp.ones((16,), jnp.float32)

        def fill(vec):
            def body(r, carry):
                rows[r, pl.ds(0, 16)] = vec
                return carry
            lax.fori_loop(0, K, body, 0)

        fill(zeros16)
        sub0 = s * RPS
        for o, n in RCH:
            pltpu.sync_copy(rows.at[pl.ds(0, n)], acc.at[pl.ds(sub0 + o, n)])
        fill(ones16)

        @pl.when(c == 0)
        def _():
            pltpu.sync_copy(dst2.at[pl.ds(s * CH0, CH0)],
                            idst.at[pl.ds(0, CH0)])

        @pl.when(c == 1)
        def _():
            pltpu.sync_copy(dst2.at[pl.ds(NS * CH0 + s * CH1, CH1)],
                            idst.at[pl.ds(0, CH1)])

        plsc.subcore_barrier()

        def grp(g, carry):
            for t in range(FD):
                i = g * FD + t

                @pl.when(i < ch)
                def _():
                    pltpu.async_copy(rows, acc.at[idst.at[i]], sem, add=True)

            for t in range(FD):
                i = g * FD + t

                @pl.when(i < ch)
                def _():
                    pltpu.make_async_copy(rows, acc.at[idst.at[i]],
                                          sem).wait()

            return carry

        lax.fori_loop(0, (CHM + FD - 1) // FD, grp, 0)
        plsc.subcore_barrier()
        for o, n in RCH:
            sl = pl.ds(sub0 + o, n)
            pltpu.sync_copy(acc.at[sl], out.at[c, sl])

    return deg


def _pre_body(cnt_ref, x_ref, dis_ref, xs_ref):
    c = cnt_ref[0, :, 0:1] + cnt_ref[1, :, 0:1] + 1.0
    d = lax.rsqrt(c)
    dis_ref[...] = d
    xs_ref[...] = d * x_ref[...]


def _l1_body(dis_ref, xs_ref, agg_ref, w1_ref, b1_ref, w2_ref, ys2_ref):
    d = dis_ref[...]
    a = d * (agg_ref[0] + agg_ref[1] + xs_ref[...])
    h1 = jnp.maximum(
        jnp.dot(a, w1_ref[...], preferred_element_type=jnp.float32)
        + b1_ref[...], 0.0)
    ys2_ref[...] = d * jnp.dot(h1, w2_ref[...],
                               preferred_element_type=jnp.float32)


def _l2_body(dis_ref, ys2_ref, agg_ref, b2_ref, w3_ref, ys3_ref):
    d = dis_ref[...]
    h2 = jnp.maximum(
        d * (agg_ref[0] + agg_ref[1] + ys2_ref[...]) + b2_ref[...], 0.0)
    ys3_ref[...] = d * jnp.dot(h2, w3_ref[...],
                               preferred_element_type=jnp.float32)


def _l3_body(dis_ref, ys3_ref, agg_ref, b3_ref, out_ref):
    d = dis_ref[...]
    o = d * (agg_ref[0] + agg_ref[1] + ys3_ref[...]) + b3_ref[...]
    lg = o[:, :10]
    m = jnp.max(lg, axis=1, keepdims=True)
    e = jnp.exp(lg - m)
    out_ref[...] = lg - m - jnp.log(jnp.sum(e, axis=1, keepdims=True))


def kernel(x, edge_index, W1, b1, W2, b2, W3, b3):
    N, D_IN = x.shape
    E = edge_index.shape[1]
    H1 = W1.shape[1]
    H2 = W2.shape[1]
    C = W3.shape[1]
    CP = 16

    K1 = 32
    K2 = 128
    NPAD = _ceil_to(N + 1, NS * 8)

    src = edge_index[0]
    dst = edge_index[1]

    def _chunked(K, F0):
        tot = -(-E // (NS * K))
        ch0 = max(2, int(round(F0 * tot / 2)) * 2)
        ch1 = max(2, -(-(tot - ch0) // 2) * 2)
        ep = NS * K * (ch0 + ch1)
        s, d = src, dst
        if ep != E:
            pad = ep - E
            s = jnp.concatenate([s, jnp.zeros((pad,), s.dtype)])
            d = jnp.concatenate([d, jnp.full((pad,), N, d.dtype)])
        return ch0, ch1, s.reshape(ep // K, K), d.reshape(ep // K, K)

    A0, A1, src2a, dst2a = _chunked(K1, 0.70)
    B0, B1, src2b, dst2b = _chunked(K2, 0.66)

    W3p = jnp.pad(W3, ((0, 0), (0, CP - C)))
    b1r = b1.reshape(1, H1)
    b2r = b2.reshape(1, H2)
    b3r = jnp.pad(b3, (0, CP - C)).reshape(1, CP)

    BN = 2000
    G = N // BN
    f32 = jnp.float32

    cnt = _sc_deg(NPAD, B0, B1, K2)(dst2b)

    dis, xs = pl.pallas_call(
        _pre_body,
        grid=(G,),
        in_specs=[
            pl.BlockSpec((NC, BN, 16), lambda i: (0, i, 0)),
            pl.BlockSpec((BN, D_IN), lambda i: (i, 0)),
        ],
        out_specs=[
            pl.BlockSpec((BN, 1), lambda i: (i, 0)),
            pl.BlockSpec((BN, D_IN), lambda i: (i, 0)),
        ],
        out_shape=[
            jax.ShapeDtypeStruct((N, 1), f32),
            jax.ShapeDtypeStruct((N, D_IN), f32),
        ],
    )(cnt, x)

    agg1 = _sc_agg(N, NPAD, A0, A1, D_IN, K1, 5)(xs, src2a, dst2a)

    ys2 = pl.pallas_call(
        _l1_body,
        grid=(G,),
        in_specs=[
            pl.BlockSpec((BN, 1), lambda i: (i, 0)),
            pl.BlockSpec((BN, D_IN), lambda i: (i, 0)),
            pl.BlockSpec((NC, BN, D_IN), lambda i: (0, i, 0)),
            pl.BlockSpec((D_IN, H1), lambda i: (0, 0)),
            pl.BlockSpec((1, H1), lambda i: (0, 0)),
            pl.BlockSpec((H1, H2), lambda i: (0, 0)),
        ],
        out_specs=pl.BlockSpec((BN, H2), lambda i: (i, 0)),
        out_shape=jax.ShapeDtypeStruct((N, H2), f32),
    )(dis, xs, agg1, W1, b1r, W2)

    agg2 = _sc_agg(N, NPAD, B0, B1, H2, K2, 6)(ys2, src2b, dst2b)

    ys3 = pl.pallas_call(
        _l2_body,
        grid=(G,),
        in_specs=[
            pl.BlockSpec((BN, 1), lambda i: (i, 0)),
            pl.BlockSpec((BN, H2), lambda i: (i, 0)),
            pl.BlockSpec((NC, BN, H2), lambda i: (0, i, 0)),
            pl.BlockSpec((1, H2), lambda i: (0, 0)),
            pl.BlockSpec((H2, CP), lambda i: (0, 0)),
        ],
        out_specs=pl.BlockSpec((BN, CP), lambda i: (i, 0)),
        out_shape=jax.ShapeDtypeStruct((N, CP), f32),
    )(dis, ys2, agg2, b2r, W3p)

    agg3 = _sc_agg(N, NPAD, B0, B1, CP, K2, 6)(ys3, src2b, dst2b)

    out = pl.pallas_call(
        _l3_body,
        grid=(G,),
        in_specs=[
            pl.BlockSpec((BN, 1), lambda i: (i, 0)),
            pl.BlockSpec((BN, CP), lambda i: (i, 0)),
            pl.BlockSpec((NC, BN, CP), lambda i: (0, i, 0)),
            pl.BlockSpec((1, CP), lambda i: (0, 0)),
        ],
        out_specs=pl.BlockSpec((BN, C), lambda i: (i, 0)),
        out_shape=jax.ShapeDtypeStruct((N, C), f32),
    )(dis, ys3, agg3, b3r)

    return out
